# baseline (speedup 1.0000x reference)
import functools

import jax
import jax.numpy as jnp
from jax import lax
from jax.experimental import pallas as pl
from jax.experimental.pallas import tpu as pltpu
from jax.experimental.pallas import tpu_sc as plsc

_INDEX = 50
_ROWS = 200
_FEAT = 64
_COLS = _ROWS * _FEAT
_COL0 = _INDEX * _FEAT

_NC = 2
_NS = 16
_NW = _NC * _NS
_B = 4096
_RPW = _B // _NW
_CH = 2
_NCHUNK = _RPW // _CH
_NBUF = 2
_LEAD = 1


def _sc_body(x_hbm, src_hbm, o_hbm, shared, srcbuf, midbuf, in_sems,
             out_sems, src_sem, mid_sem):
    sid = lax.axis_index("s")
    wid = sid * _NC + lax.axis_index("c")
    base = wid * _RPW

    src_in = pltpu.make_async_copy(
        src_hbm.at[pl.ds(base, _RPW)], srcbuf, src_sem)
    src_in.start()
    mid_in = pltpu.make_async_copy(
        x_hbm.at[pl.ds(base, _RPW), pl.ds(_COL0, 128)], midbuf, mid_sem)
    mid_in.start()

    in_copy = [
        pltpu.make_async_copy(
            x_hbm.at[pl.ds(base + i * _CH, _CH)],
            shared.at[sid, i % _NBUF], in_sems.at[i % _NBUF])
        for i in range(_NCHUNK)
    ]
    out_copy = [
        pltpu.make_async_copy(
            shared.at[sid, i % _NBUF],
            o_hbm.at[pl.ds(base + i * _CH, _CH)], out_sems.at[i % _NBUF])
        for i in range(_NCHUNK)
    ]

    for i in range(_LEAD):
        in_copy[i].start()
    waited = set()
    for i in range(_NCHUNK):
        j = i + _LEAD
        if j < _NCHUNK:
            if j >= _NBUF:
                out_copy[j - _NBUF].wait()
                waited.add(j - _NBUF)
            in_copy[j].start()
        in_copy[i].wait()
        out_copy[i].start()
    for i in range(_NCHUNK):
        if i not in waited:
            out_copy[i].wait()

    src_in.wait()
    mid_in.wait()
    for r in range(_RPW):
        for v in range(_FEAT // 16):
            midbuf[r, pl.ds(v * 16, 16)] = srcbuf[r, pl.ds(v * 16, 16)]
    mid_out = pltpu.make_async_copy(
        midbuf, o_hbm.at[pl.ds(base, _RPW), pl.ds(_COL0, 128)], src_sem)
    mid_out.start()
    mid_out.wait()


def kernel(x, src):
    b = x.shape[0]
    x2 = x.reshape(b, _COLS)
    mesh = plsc.VectorSubcoreMesh(core_axis_name="c", subcore_axis_name="s")
    run = functools.partial(
        pl.kernel,
        mesh=mesh,
        out_type=jax.ShapeDtypeStruct((b, _COLS), x.dtype),
        scratch_types=[
            pltpu.VMEM_SHARED((_NS, _NBUF, _CH, _COLS), x.dtype),
            pltpu.VMEM((_RPW, _FEAT), x.dtype),
            pltpu.VMEM((_RPW, 128), x.dtype),
            pltpu.SemaphoreType.DMA((_NBUF,)),
            pltpu.SemaphoreType.DMA((_NBUF,)),
            pltpu.SemaphoreType.DMA,
            pltpu.SemaphoreType.DMA,
        ],
    )(_sc_body)
    out = run(x2, src)
    return out.reshape(x.shape)
